# baseline (device time: 1300637 ns/iter reference)
import jax
import jax.numpy as jnp
from jax import lax
from jax.experimental import pallas as pl
from jax.experimental.pallas import tpu as pltpu

N_DEV = 4
S = 4


def kernel(x):
    m, n = x.shape
    m_ch = m // N_DEV
    n_half = n // 2
    m_sub = m_ch // S
    R = N_DEV - 1
    H = 2 * R

    def body(x_ref, out_ref, buf, x_stage, send_sems, recv_sems, copy_sems, out_sems):
        my = lax.axis_index("i")
        nbr = [jnp.mod(my + 1, N_DEV), jnp.mod(my - 1, N_DEV)]
        sgn = [1, -1]
        cols = [pl.ds(0, n_half), pl.ds(n_half, n_half)]

        def sub_rows(c, b):
            return pl.ds(c * m_ch + b * m_sub, m_sub)

        def send_chunk(h, d):
            if h < R:
                return jnp.mod(my - sgn[d] * h, N_DEV)
            return jnp.mod(my - sgn[d] * (h - R - 1), N_DEV)

        def recv_chunk(h, d):
            if h < R:
                return jnp.mod(my - sgn[d] * (h + 1), N_DEV)
            return jnp.mod(my - sgn[d] * (h - R), N_DEV)

        def make_send(h, d, b):
            c = send_chunk(h, d)
            src_ref = x_ref if h == 0 else buf
            return pltpu.make_async_remote_copy(
                src_ref=src_ref.at[sub_rows(c, b), cols[d]],
                dst_ref=buf.at[sub_rows(c, b), cols[d]],
                send_sem=send_sems.at[d, h, b],
                recv_sem=recv_sems.at[d, h, b],
                device_id=(nbr[d],),
                device_id_type=pl.DeviceIdType.MESH,
            )

        def make_recv(h, d, b):
            c = recv_chunk(h, d)
            return pltpu.make_async_remote_copy(
                src_ref=buf.at[sub_rows(c, b), cols[d]],
                dst_ref=buf.at[sub_rows(c, b), cols[d]],
                send_sem=send_sems.at[d, h, b],
                recv_sem=recv_sems.at[d, h, b],
                device_id=(nbr[d],),
                device_id_type=pl.DeviceIdType.MESH,
            )

        out_copies = []

        def flush(c, d, b, g):
            cp = pltpu.make_async_copy(
                buf.at[sub_rows(c, b), cols[d]],
                out_ref.at[sub_rows(c, b), cols[d]],
                out_sems.at[d, g - (R - 1), b],
            )
            cp.start()
            out_copies.append(cp)

        copies = []
        for d in range(2):
            per_d = []
            for g in range(R):
                cp = pltpu.make_async_copy(
                    x_ref.at[pl.ds(recv_chunk(g, d) * m_ch, m_ch), cols[d]],
                    x_stage.at[d, g],
                    copy_sems.at[d, g],
                )
                cp.start()
                per_d.append(cp)
            copies.append(per_d)

        started = []
        for b in range(S):
            for d in range(2):
                rdma = make_send(0, d, b)
                rdma.start()
                started.append(rdma)

        for h in range(1, H + 1):
            g = h - 1
            if g < R:
                for d in range(2):
                    copies[d][g].wait()
            for b in range(S):
                for d in range(2):
                    make_recv(g, d, b).wait_recv()
                    if g < R:
                        rc = recv_chunk(g, d)
                        buf[sub_rows(rc, b), cols[d]] = (
                            buf[sub_rows(rc, b), cols[d]]
                            + x_stage[d, g, pl.ds(b * m_sub, m_sub), :]
                        )
                        if g == R - 1:
                            flush(rc, d, b, g)
                    else:
                        flush(recv_chunk(g, d), d, b, g)
                    if h < H:
                        rdma = make_send(h, d, b)
                        rdma.start()
                        started.append(rdma)

        for cp in out_copies:
            cp.wait()
        for rdma in started:
            rdma.wait_send()

    ring_out = pl.pallas_call(
        body,
        out_shape=jax.ShapeDtypeStruct((m, n), x.dtype),
        in_specs=[pl.BlockSpec(memory_space=pl.ANY)],
        out_specs=pl.BlockSpec(memory_space=pl.ANY),
        scratch_shapes=[
            pltpu.VMEM((m, n), x.dtype),
            pltpu.VMEM((2, R, m_ch, n_half), x.dtype),
            pltpu.SemaphoreType.DMA((2, H, S)),
            pltpu.SemaphoreType.DMA((2, H, S)),
            pltpu.SemaphoreType.DMA((2, R)),
            pltpu.SemaphoreType.DMA((2, N_DEV, S)),
        ],
        compiler_params=pltpu.CompilerParams(
            vmem_limit_bytes=60 * 1024 * 1024,
        ),
    )(x)
    return _materialize(ring_out)


def _materialize(y):
    m, n = y.shape
    n_dma = 8
    m_blk = m // n_dma

    def body(y_ref, o_ref, sems):
        cps = []
        for i in range(n_dma):
            cp = pltpu.make_async_copy(
                y_ref.at[pl.ds(i * m_blk, m_blk), :],
                o_ref.at[pl.ds(i * m_blk, m_blk), :],
                sems.at[i],
            )
            cp.start()
            cps.append(cp)
        for cp in cps:
            cp.wait()

    return pl.pallas_call(
        body,
        out_shape=jax.ShapeDtypeStruct((m, n), y.dtype),
        in_specs=[pl.BlockSpec(memory_space=pl.ANY)],
        out_specs=pl.BlockSpec(memory_space=pl.ANY),
        scratch_shapes=[pltpu.SemaphoreType.DMA((n_dma,))],
    )(y)


# device time: 305111 ns/iter; 4.2628x vs baseline; 4.2628x over previous
import jax
import jax.numpy as jnp
from jax import lax
from jax.experimental import pallas as pl
from jax.experimental.pallas import tpu as pltpu

N_DEV = 4
S = 4


def kernel(x):
    m, n = x.shape
    m_ch = m // N_DEV
    n_half = n // 2
    m_sub = m_ch // S
    R = N_DEV - 1
    H = 2 * R

    def body(x_ref, out_ref, buf, x_stage, send_sems, recv_sems, copy_sems, out_sems):
        my = lax.axis_index("i")
        nbr = [jnp.mod(my + 1, N_DEV), jnp.mod(my - 1, N_DEV)]
        sgn = [1, -1]
        cols = [pl.ds(0, n_half), pl.ds(n_half, n_half)]

        def sub_rows(c, b):
            return pl.ds(c * m_ch + b * m_sub, m_sub)

        def send_chunk(h, d):
            if h < R:
                return jnp.mod(my - sgn[d] * h, N_DEV)
            return jnp.mod(my - sgn[d] * (h - R - 1), N_DEV)

        def recv_chunk(h, d):
            if h < R:
                return jnp.mod(my - sgn[d] * (h + 1), N_DEV)
            return jnp.mod(my - sgn[d] * (h - R), N_DEV)

        def make_send(h, d, b):
            c = send_chunk(h, d)
            src_ref = x_ref if h == 0 else buf
            return pltpu.make_async_remote_copy(
                src_ref=src_ref.at[sub_rows(c, b), cols[d]],
                dst_ref=buf.at[sub_rows(c, b), cols[d]],
                send_sem=send_sems.at[d, h, b],
                recv_sem=recv_sems.at[d, h, b],
                device_id=(nbr[d],),
                device_id_type=pl.DeviceIdType.MESH,
            )

        def make_recv(h, d, b):
            c = recv_chunk(h, d)
            return pltpu.make_async_remote_copy(
                src_ref=buf.at[sub_rows(c, b), cols[d]],
                dst_ref=buf.at[sub_rows(c, b), cols[d]],
                send_sem=send_sems.at[d, h, b],
                recv_sem=recv_sems.at[d, h, b],
                device_id=(nbr[d],),
                device_id_type=pl.DeviceIdType.MESH,
            )

        out_copies = []

        def flush(c, d, b, g):
            cp = pltpu.make_async_copy(
                buf.at[sub_rows(c, b), cols[d]],
                out_ref.at[sub_rows(c, b), cols[d]],
                out_sems.at[d, g - (R - 1), b],
            )
            cp.start()
            out_copies.append(cp)

        copies = []
        for d in range(2):
            per_d = []
            for g in range(R):
                cp = pltpu.make_async_copy(
                    x_ref.at[pl.ds(recv_chunk(g, d) * m_ch, m_ch), cols[d]],
                    x_stage.at[d, g],
                    copy_sems.at[d, g],
                )
                cp.start()
                per_d.append(cp)
            copies.append(per_d)

        started = []
        for b in range(S):
            for d in range(2):
                rdma = make_send(0, d, b)
                rdma.start()
                started.append(rdma)

        for h in range(1, H + 1):
            g = h - 1
            if g < R:
                for d in range(2):
                    copies[d][g].wait()
            for b in range(S):
                for d in range(2):
                    make_recv(g, d, b).wait_recv()
                    if g < R:
                        rc = recv_chunk(g, d)
                        buf[sub_rows(rc, b), cols[d]] = (
                            buf[sub_rows(rc, b), cols[d]]
                            + x_stage[d, g, pl.ds(b * m_sub, m_sub), :]
                        )
                        if g == R - 1:
                            flush(rc, d, b, g)
                    else:
                        flush(recv_chunk(g, d), d, b, g)
                    if h < H:
                        rdma = make_send(h, d, b)
                        rdma.start()
                        started.append(rdma)

        for cp in out_copies:
            cp.wait()
        for rdma in started:
            rdma.wait_send()

    ring_out = pl.pallas_call(
        body,
        out_shape=jax.ShapeDtypeStruct((m, n), x.dtype),
        in_specs=[pl.BlockSpec(memory_space=pl.ANY)],
        out_specs=pl.BlockSpec(memory_space=pl.ANY),
        scratch_shapes=[
            pltpu.VMEM((m, n), x.dtype),
            pltpu.VMEM((2, R, m_ch, n_half), x.dtype),
            pltpu.SemaphoreType.DMA((2, H, S)),
            pltpu.SemaphoreType.DMA((2, H, S)),
            pltpu.SemaphoreType.DMA((2, R)),
            pltpu.SemaphoreType.DMA((2, N_DEV, S)),
        ],
        compiler_params=pltpu.CompilerParams(
            vmem_limit_bytes=60 * 1024 * 1024,
        ),
    )(x)
    return _materialize(ring_out)


def _materialize(y):
    m, n = y.shape
    blk = 512

    def body(y_ref, o_ref):
        o_ref[...] = y_ref[...]

    return pl.pallas_call(
        body,
        out_shape=jax.ShapeDtypeStruct((m, n), y.dtype),
        in_specs=[pl.BlockSpec((blk, n), lambda i: (i, 0))],
        out_specs=pl.BlockSpec((blk, n), lambda i: (i, 0)),
        grid=(m // blk,),
    )(y)


# device time: 299669 ns/iter; 4.3402x vs baseline; 1.0182x over previous
import jax
import jax.numpy as jnp
from jax import lax
from jax.experimental import pallas as pl
from jax.experimental.pallas import tpu as pltpu

N_DEV = 4
S = 4


def kernel(x):
    m, n = x.shape
    m_ch = m // N_DEV
    n_half = n // 2
    m_sub = m_ch // S
    R = N_DEV - 1
    H = 2 * R

    def body(x_ref, out_ref, buf, x_stage, send_sems, recv_sems, copy_sems, out_sems):
        my = lax.axis_index("i")
        nbr = [jnp.mod(my + 1, N_DEV), jnp.mod(my - 1, N_DEV)]
        sgn = [1, -1]
        cols = [pl.ds(0, n_half), pl.ds(n_half, n_half)]

        def sub_rows(c, b):
            return pl.ds(c * m_ch + b * m_sub, m_sub)

        def send_chunk(h, d):
            if h < R:
                return jnp.mod(my - sgn[d] * h, N_DEV)
            return jnp.mod(my - sgn[d] * (h - R - 1), N_DEV)

        def recv_chunk(h, d):
            if h < R:
                return jnp.mod(my - sgn[d] * (h + 1), N_DEV)
            return jnp.mod(my - sgn[d] * (h - R), N_DEV)

        def make_send(h, d, b):
            c = send_chunk(h, d)
            src_ref = x_ref if h == 0 else buf
            return pltpu.make_async_remote_copy(
                src_ref=src_ref.at[sub_rows(c, b), cols[d]],
                dst_ref=buf.at[sub_rows(c, b), cols[d]],
                send_sem=send_sems.at[d, h, b],
                recv_sem=recv_sems.at[d, h, b],
                device_id=(nbr[d],),
                device_id_type=pl.DeviceIdType.MESH,
            )

        def make_recv(h, d, b):
            c = recv_chunk(h, d)
            return pltpu.make_async_remote_copy(
                src_ref=buf.at[sub_rows(c, b), cols[d]],
                dst_ref=buf.at[sub_rows(c, b), cols[d]],
                send_sem=send_sems.at[d, h, b],
                recv_sem=recv_sems.at[d, h, b],
                device_id=(nbr[d],),
                device_id_type=pl.DeviceIdType.MESH,
            )

        barrier_sem = pltpu.get_barrier_semaphore()
        for d in range(2):
            pl.semaphore_signal(
                barrier_sem,
                inc=1,
                device_id=(nbr[d],),
                device_id_type=pl.DeviceIdType.MESH,
            )
        pl.semaphore_wait(barrier_sem, 2)

        out_copies = []

        def flush(c, d, b, g):
            cp = pltpu.make_async_copy(
                buf.at[sub_rows(c, b), cols[d]],
                out_ref.at[sub_rows(c, b), cols[d]],
                out_sems.at[d, g - (R - 1), b],
            )
            cp.start()
            out_copies.append(cp)

        copies = []
        for d in range(2):
            per_d = []
            for g in range(R):
                cp = pltpu.make_async_copy(
                    x_ref.at[pl.ds(recv_chunk(g, d) * m_ch, m_ch), cols[d]],
                    x_stage.at[d, g],
                    copy_sems.at[d, g],
                )
                cp.start()
                per_d.append(cp)
            copies.append(per_d)

        started = []
        for b in range(S):
            for d in range(2):
                rdma = make_send(0, d, b)
                rdma.start()
                started.append(rdma)

        for h in range(1, H + 1):
            g = h - 1
            if g < R:
                for d in range(2):
                    copies[d][g].wait()
            for b in range(S):
                for d in range(2):
                    make_recv(g, d, b).wait_recv()
                    if g < R:
                        rc = recv_chunk(g, d)
                        buf[sub_rows(rc, b), cols[d]] = (
                            buf[sub_rows(rc, b), cols[d]]
                            + x_stage[d, g, pl.ds(b * m_sub, m_sub), :]
                        )
                        if g == R - 1:
                            flush(rc, d, b, g)
                    else:
                        flush(recv_chunk(g, d), d, b, g)
                    if h < H:
                        rdma = make_send(h, d, b)
                        rdma.start()
                        started.append(rdma)

        for cp in out_copies:
            cp.wait()
        for rdma in started:
            rdma.wait_send()

    ring_out = pl.pallas_call(
        body,
        out_shape=jax.ShapeDtypeStruct((m, n), x.dtype),
        in_specs=[pl.BlockSpec(memory_space=pl.ANY)],
        out_specs=pl.BlockSpec(memory_space=pl.ANY),
        scratch_shapes=[
            pltpu.VMEM((m, n), x.dtype),
            pltpu.VMEM((2, R, m_ch, n_half), x.dtype),
            pltpu.SemaphoreType.DMA((2, H, S)),
            pltpu.SemaphoreType.DMA((2, H, S)),
            pltpu.SemaphoreType.DMA((2, R)),
            pltpu.SemaphoreType.DMA((2, N_DEV, S)),
        ],
        compiler_params=pltpu.CompilerParams(
            vmem_limit_bytes=60 * 1024 * 1024,
            collective_id=0,
        ),
    )(x)
    return ring_out


def _materialize(y):
    m, n = y.shape
    blk = 512

    def body(y_ref, o_ref):
        o_ref[...] = y_ref[...]

    return pl.pallas_call(
        body,
        out_shape=jax.ShapeDtypeStruct((m, n), y.dtype),
        in_specs=[pl.BlockSpec((blk, n), lambda i: (i, 0))],
        out_specs=pl.BlockSpec((blk, n), lambda i: (i, 0)),
        grid=(m // blk,),
    )(y)
